# Initial kernel scaffold; baseline (speedup 1.0000x reference)
import jax
import jax.numpy as jnp
from jax.experimental import pallas as pl


def kernel(x_input_node, edge_index, edge_input_attr, A_w, A_b, B_w, B_b, C_w, C_b, D_w, D_b, E_w, E_b, res_e_w):
    raise NotImplementedError("write your pallas kernel here")


# speedup vs baseline: 1.2217x; 1.2217x over previous
import functools

import jax
import jax.numpy as jnp
from jax import lax
from jax.experimental import pallas as pl
from jax.experimental.pallas import tpu as pltpu
from jax.experimental.pallas import tpu_sc as plsc

N = 10000
E = 320000
DN = 128
DE = 16
DO = 128


_NODE_BLK = 1000


def _nodeproj_body(x_ref, w_ref, b_ref, ax_ref, dx_ref, exbx_ref):
    h = lax.dot_general(x_ref[...], w_ref[...], (((1,), (1,)), ((), ())),
                        preferred_element_type=jnp.float32)
    h = h + b_ref[...]
    ax_ref[...] = h[:, :DO]
    dx_ref[...] = h[:, DO:2 * DO]
    exbx_ref[...] = h[:, 2 * DO:]


def _nodeproj(x, w_cat, b_cat):
    grid = N // _NODE_BLK
    return pl.pallas_call(
        _nodeproj_body,
        grid=(grid,),
        in_specs=[
            pl.BlockSpec((_NODE_BLK, DN), lambda i: (i, 0)),
            pl.BlockSpec((4 * DO, DN), lambda i: (0, 0)),
            pl.BlockSpec((1, 4 * DO), lambda i: (0, 0)),
        ],
        out_specs=[
            pl.BlockSpec((_NODE_BLK, DO), lambda i: (i, 0)),
            pl.BlockSpec((_NODE_BLK, DO), lambda i: (i, 0)),
            pl.BlockSpec((_NODE_BLK, 2 * DO), lambda i: (i, 0)),
        ],
        out_shape=[
            jax.ShapeDtypeStruct((N, DO), jnp.float32),
            jax.ShapeDtypeStruct((N, DO), jnp.float32),
            jax.ShapeDtypeStruct((N, 2 * DO), jnp.float32),
        ],
    )(x, w_cat, b_cat)


_EDGE_BLK = 8000


def _edgeproj_body(a_ref, w_ref, b_ref, o_ref):
    o_ref[...] = lax.dot_general(a_ref[...], w_ref[...], (((1,), (1,)), ((), ())),
                                 preferred_element_type=jnp.float32) + b_ref[...]


def _edgeproj(attr, w_cat, b_cat):
    grid = E // _EDGE_BLK
    return pl.pallas_call(
        _edgeproj_body,
        grid=(grid,),
        in_specs=[
            pl.BlockSpec((_EDGE_BLK, DE), lambda i: (i, 0)),
            pl.BlockSpec((2 * DO, DE), lambda i: (0, 0)),
            pl.BlockSpec((1, 2 * DO), lambda i: (0, 0)),
        ],
        out_specs=pl.BlockSpec((_EDGE_BLK, 2 * DO), lambda i: (i, 0)),
        out_shape=jax.ShapeDtypeStruct((E, 2 * DO), jnp.float32),
    )(attr, w_cat, b_cat)


_NW = 32
_E_PER_W = E // _NW
_CHUNK = 40
_NCHUNK = _E_PER_W // _CHUNK
_ROWS_PER_TILE = 624
_TAIL_LO = 16 * _ROWS_PER_TILE
_TAIL = N - _TAIL_LO


def _sc_edge_body(row_hbm, col_hbm, dx_hbm, exbx_hbm, cer_hbm,
                  eout_hbm, aggr_hbm,
                  row_v, col_v, dxg, exbxg, cer_v, eout_v, msg_v,
                  acc_sh, sem1, sem2):
    cid = lax.axis_index("c")
    sid = lax.axis_index("s")
    wid = sid * 2 + cid
    base = wid * _E_PER_W

    zero16 = jnp.zeros((16,), jnp.float32)

    def zfill(i, _):
        for j in range(DO // 16):
            msg_v[i, pl.ds(j * 16, 16)] = zero16
        return 0
    lax.fori_loop(0, _CHUNK, zfill, 0)
    for z in range(16):
        pltpu.sync_copy(msg_v, acc_sh.at[pl.ds(sid * _ROWS_PER_TILE + z * _CHUNK, _CHUNK)])

    plsc.subcore_barrier()

    def chunk_body(k, _):
        start = base + k * _CHUNK
        pltpu.sync_copy(row_hbm.at[pl.ds(start, _CHUNK)], row_v)
        pltpu.sync_copy(col_hbm.at[pl.ds(start, _CHUNK)], col_v)
        cp1 = pltpu.async_copy(dx_hbm.at[row_v], dxg, sem1)
        cp2 = pltpu.async_copy(exbx_hbm.at[col_v], exbxg, sem2)
        pltpu.sync_copy(cer_hbm.at[pl.ds(start, _CHUNK)], cer_v)
        cp1.wait()
        cp2.wait()

        def edge_body(i, _):
            for j in range(DO // 16):
                sl = pl.ds(j * 16, 16)
                sl2 = pl.ds(DO + j * 16, 16)
                e = dxg[i, sl] + exbxg[i, sl] + cer_v[i, sl]
                eout_v[i, sl] = cer_v[i, sl2] + jnp.maximum(e, 0.0)
                sig = 1.0 / (1.0 + jnp.exp(-e))
                msg_v[i, sl] = sig * exbxg[i, sl2]
            return 0
        lax.fori_loop(0, _CHUNK, edge_body, 0)

        pltpu.sync_copy(eout_v, eout_hbm.at[pl.ds(start, _CHUNK)])
        pltpu.sync_copy(msg_v, acc_sh.at[row_v], add=True)
        return 0

    lax.fori_loop(0, _NCHUNK, chunk_body, 0)
    plsc.subcore_barrier()

    lo = sid * _ROWS_PER_TILE
    pltpu.sync_copy(acc_sh.at[pl.ds(lo, _ROWS_PER_TILE)],
                    aggr_hbm.at[cid, pl.ds(lo, _ROWS_PER_TILE)])

    @pl.when(sid == 15)
    def _copy_tail():
        pltpu.sync_copy(acc_sh.at[pl.ds(_TAIL_LO, _TAIL)],
                        aggr_hbm.at[cid, pl.ds(_TAIL_LO, _TAIL)])


def _sc_edge(row, col, dx, exbx, cer):
    mesh = plsc.VectorSubcoreMesh(core_axis_name="c", subcore_axis_name="s")
    fn = functools.partial(
        pl.kernel,
        mesh=mesh,
        out_type=[
            jax.ShapeDtypeStruct((E, DO), jnp.float32),
            jax.ShapeDtypeStruct((2, N, DO), jnp.float32),
        ],
        scratch_types=[
            pltpu.VMEM((_CHUNK,), jnp.int32),
            pltpu.VMEM((_CHUNK,), jnp.int32),
            pltpu.VMEM((_CHUNK, DO), jnp.float32),
            pltpu.VMEM((_CHUNK, 2 * DO), jnp.float32),
            pltpu.VMEM((_CHUNK, 2 * DO), jnp.float32),
            pltpu.VMEM((_CHUNK, DO), jnp.float32),
            pltpu.VMEM((_CHUNK, DO), jnp.float32),
            pltpu.VMEM_SHARED((N, DO), jnp.float32),
            pltpu.SemaphoreType.DMA,
            pltpu.SemaphoreType.DMA,
        ],
    )(_sc_edge_body)
    return fn(row, col, dx, exbx, cer)


def _final_body(x_ref, ax_ref, ag_ref, o_ref):
    o_ref[...] = x_ref[...] + jnp.maximum(
        ax_ref[...] + ag_ref[0] + ag_ref[1], 0.0)


def _final(x, ax, aggr):
    grid = N // _NODE_BLK
    return pl.pallas_call(
        _final_body,
        grid=(grid,),
        in_specs=[
            pl.BlockSpec((_NODE_BLK, DO), lambda i: (i, 0)),
            pl.BlockSpec((_NODE_BLK, DO), lambda i: (i, 0)),
            pl.BlockSpec((2, _NODE_BLK, DO), lambda i: (0, i, 0)),
        ],
        out_specs=pl.BlockSpec((_NODE_BLK, DO), lambda i: (i, 0)),
        out_shape=jax.ShapeDtypeStruct((N, DO), jnp.float32),
    )(x, ax, aggr)


def kernel(x_input_node, edge_index, edge_input_attr,
           A_w, A_b, B_w, B_b, C_w, C_b, D_w, D_b, E_w, E_b, res_e_w):
    row = edge_index[0].astype(jnp.int32)
    col = edge_index[1].astype(jnp.int32)

    w_node = jnp.concatenate([A_w, D_w, E_w, B_w], axis=0)
    b_node = jnp.concatenate([A_b, D_b, E_b, B_b], axis=0)[None]
    w_edge = jnp.concatenate([C_w, res_e_w], axis=0)
    b_edge = jnp.concatenate([C_b, jnp.zeros_like(C_b)], axis=0)[None]

    ax, dx, exbx = _nodeproj(x_input_node, w_node, b_node)
    cer = _edgeproj(edge_input_attr, w_edge, b_edge)
    eout, aggr = _sc_edge(row, col, dx, exbx, cer)
    x_final = _final(x_input_node, ax, aggr)
    return (x_final, eout)


# speedup vs baseline: 1.3988x; 1.1449x over previous
import functools

import jax
import jax.numpy as jnp
from jax import lax
from jax.experimental import pallas as pl
from jax.experimental.pallas import tpu as pltpu
from jax.experimental.pallas import tpu_sc as plsc

N = 10000
E = 320000
DN = 128
DE = 16
DO = 128


_NODE_BLK = 1000


def _nodeproj_body(x_ref, w_ref, b_ref, ax_ref, dx_ref, exbx_ref):
    h = lax.dot_general(x_ref[...], w_ref[...], (((1,), (1,)), ((), ())),
                        preferred_element_type=jnp.float32)
    h = h + b_ref[...]
    ax_ref[...] = h[:, :DO]
    dx_ref[...] = h[:, DO:2 * DO]
    exbx_ref[...] = h[:, 2 * DO:]


def _nodeproj(x, w_cat, b_cat):
    grid = N // _NODE_BLK
    return pl.pallas_call(
        _nodeproj_body,
        grid=(grid,),
        in_specs=[
            pl.BlockSpec((_NODE_BLK, DN), lambda i: (i, 0)),
            pl.BlockSpec((4 * DO, DN), lambda i: (0, 0)),
            pl.BlockSpec((1, 4 * DO), lambda i: (0, 0)),
        ],
        out_specs=[
            pl.BlockSpec((_NODE_BLK, DO), lambda i: (i, 0)),
            pl.BlockSpec((_NODE_BLK, DO), lambda i: (i, 0)),
            pl.BlockSpec((_NODE_BLK, 2 * DO), lambda i: (i, 0)),
        ],
        out_shape=[
            jax.ShapeDtypeStruct((N, DO), jnp.float32),
            jax.ShapeDtypeStruct((N, DO), jnp.float32),
            jax.ShapeDtypeStruct((N, 2 * DO), jnp.float32),
        ],
    )(x, w_cat, b_cat)


_EDGE_BLK = 8000


def _edgeproj_body(a_ref, w_ref, b_ref, o_ref):
    o_ref[...] = lax.dot_general(a_ref[...], w_ref[...], (((1,), (1,)), ((), ())),
                                 preferred_element_type=jnp.float32) + b_ref[...]


def _edgeproj(attr, w_cat, b_cat):
    grid = E // _EDGE_BLK
    return pl.pallas_call(
        _edgeproj_body,
        grid=(grid,),
        in_specs=[
            pl.BlockSpec((_EDGE_BLK, DE), lambda i: (i, 0)),
            pl.BlockSpec((2 * DO, DE), lambda i: (0, 0)),
            pl.BlockSpec((1, 2 * DO), lambda i: (0, 0)),
        ],
        out_specs=pl.BlockSpec((_EDGE_BLK, 2 * DO), lambda i: (i, 0)),
        out_shape=jax.ShapeDtypeStruct((E, 2 * DO), jnp.float32),
    )(attr, w_cat, b_cat)


_NW = 32
_E_PER_W = E // _NW
_CK = 16
_NCH = _E_PER_W // _CK
_ROWS_PER_TILE = 624
_TAIL_LO = 16 * _ROWS_PER_TILE
_TAIL = N - _TAIL_LO


def _sc_edge_body(row_hbm, col_hbm, dx_hbm, exbx_hbm, cer_hbm,
                  eout_hbm, aggr_hbm,
                  row_sc0, row_sc1, col_sc0, col_sc1,
                  dxg, exbxg, cer_v, eout_v, msg_v,
                  acc_sh, isem, gsem, osem):
    cid = lax.axis_index("c")
    sid = lax.axis_index("s")
    wid = sid * 2 + cid
    base = wid * _E_PER_W
    row_sc = (row_sc0, row_sc1)
    col_sc = (col_sc0, col_sc1)

    zero16 = jnp.zeros((16,), jnp.float32)

    def zfill(i, _):
        for j in range(DO // 16):
            msg_v[i, pl.ds(j * 16, 16)] = zero16
        return 0
    lax.fori_loop(0, _CK, zfill, 0)
    for z in range(40):
        pltpu.sync_copy(msg_v, acc_sh.at[pl.ds(sid * _ROWS_PER_TILE + z * _CK, _CK)])

    plsc.subcore_barrier()


    def issue_idx(k, b):
        kc = jnp.minimum(k, _NCH - 1)
        start = base + kc * _CK
        pltpu.async_copy(row_hbm.at[pl.ds(start, _CK)], row_sc[b], isem)
        pltpu.async_copy(col_hbm.at[pl.ds(start, _CK)], col_sc[b], isem)

    def wait_idx(b):
        pltpu.make_async_copy(row_hbm.at[pl.ds(0, _CK)], row_sc[b], isem).wait()
        pltpu.make_async_copy(col_hbm.at[pl.ds(0, _CK)], col_sc[b], isem).wait()

    def issue_in(k, b):
        start = base + k * _CK
        pltpu.async_copy(dx_hbm.at[row_sc[b]], dxg.at[b], gsem)
        pltpu.async_copy(exbx_hbm.at[col_sc[b]], exbxg.at[b], gsem)
        pltpu.async_copy(cer_hbm.at[pl.ds(start, _CK)], cer_v.at[b], gsem)

    def wait_in(b):
        pltpu.make_async_copy(dx_hbm.at[pl.ds(0, _CK)], dxg.at[b], gsem).wait()
        pltpu.make_async_copy(exbx_hbm.at[pl.ds(0, _CK)], exbxg.at[b], gsem).wait()
        pltpu.make_async_copy(cer_hbm.at[pl.ds(0, _CK)], cer_v.at[b], gsem).wait()

    def wait_out():
        pltpu.make_async_copy(eout_hbm.at[pl.ds(0, _CK)], eout_v, osem).wait()

    def compute(b):
        def edge_body(i, _):
            for j in range(DO // 16):
                sl = pl.ds(j * 16, 16)
                sl2 = pl.ds(DO + j * 16, 16)
                e = dxg[b, i, sl] + exbxg[b, i, sl] + cer_v[b, i, sl]
                eout_v[i, sl] = cer_v[b, i, sl2] + jnp.maximum(e, 0.0)
                sig = 1.0 / (1.0 + jnp.exp(-e))
                msg_v[i, sl] = sig * exbxg[b, i, sl2]
            return 0
        lax.fori_loop(0, _CK, edge_body, 0)

    issue_idx(0, 0)
    issue_idx(1, 1)
    wait_idx(0)
    issue_in(0, 0)
    pltpu.async_copy(eout_hbm.at[pl.ds(0, _CK)], eout_v, osem)

    def step(k, b):
        wait_in(b)
        wait_out()
        wait_idx(1 - b)
        issue_in(k + 1, 1 - b)
        compute(b)
        start = base + k * _CK
        pltpu.async_copy(eout_v, eout_hbm.at[pl.ds(start, _CK)], osem)
        pltpu.sync_copy(msg_v, acc_sh.at[row_sc[b]], add=True)
        issue_idx(k + 2, b)

    def pair_body(g, _):
        step(2 * g, 0)
        step(2 * g + 1, 1)
        return 0
    lax.fori_loop(0, (_NCH - 1) // 2, pair_body, 0)

    wait_in(0)
    wait_out()
    compute(0)
    pltpu.async_copy(eout_v, eout_hbm.at[pl.ds(base + (_NCH - 1) * _CK, _CK)],
                     osem)
    pltpu.sync_copy(msg_v, acc_sh.at[row_sc0], add=True)
    wait_out()
    wait_idx(1)

    plsc.subcore_barrier()

    lo = sid * _ROWS_PER_TILE
    pltpu.sync_copy(acc_sh.at[pl.ds(lo, _ROWS_PER_TILE)],
                    aggr_hbm.at[cid, pl.ds(lo, _ROWS_PER_TILE)])

    @pl.when(sid == 15)
    def _copy_tail():
        pltpu.sync_copy(acc_sh.at[pl.ds(_TAIL_LO, _TAIL)],
                        aggr_hbm.at[cid, pl.ds(_TAIL_LO, _TAIL)])


def _sc_edge(row, col, dx, exbx, cer):
    mesh = plsc.VectorSubcoreMesh(core_axis_name="c", subcore_axis_name="s")
    fn = functools.partial(
        pl.kernel,
        mesh=mesh,
        out_type=[
            jax.ShapeDtypeStruct((E, DO), jnp.float32),
            jax.ShapeDtypeStruct((2, N, DO), jnp.float32),
        ],
        scratch_types=[
            pltpu.VMEM((_CK,), jnp.int32),
            pltpu.VMEM((_CK,), jnp.int32),
            pltpu.VMEM((_CK,), jnp.int32),
            pltpu.VMEM((_CK,), jnp.int32),
            pltpu.VMEM((2, _CK, DO), jnp.float32),
            pltpu.VMEM((2, _CK, 2 * DO), jnp.float32),
            pltpu.VMEM((2, _CK, 2 * DO), jnp.float32),
            pltpu.VMEM((_CK, DO), jnp.float32),
            pltpu.VMEM((_CK, DO), jnp.float32),
            pltpu.VMEM_SHARED((N, DO), jnp.float32),
            pltpu.SemaphoreType.DMA,
            pltpu.SemaphoreType.DMA,
            pltpu.SemaphoreType.DMA,
        ],
    )(_sc_edge_body)
    return fn(row, col, dx, exbx, cer)


def _final_body(x_ref, ax_ref, ag_ref, o_ref):
    o_ref[...] = x_ref[...] + jnp.maximum(
        ax_ref[...] + ag_ref[0] + ag_ref[1], 0.0)


def _final(x, ax, aggr):
    grid = N // _NODE_BLK
    return pl.pallas_call(
        _final_body,
        grid=(grid,),
        in_specs=[
            pl.BlockSpec((_NODE_BLK, DO), lambda i: (i, 0)),
            pl.BlockSpec((_NODE_BLK, DO), lambda i: (i, 0)),
            pl.BlockSpec((2, _NODE_BLK, DO), lambda i: (0, i, 0)),
        ],
        out_specs=pl.BlockSpec((_NODE_BLK, DO), lambda i: (i, 0)),
        out_shape=jax.ShapeDtypeStruct((N, DO), jnp.float32),
    )(x, ax, aggr)


def kernel(x_input_node, edge_index, edge_input_attr,
           A_w, A_b, B_w, B_b, C_w, C_b, D_w, D_b, E_w, E_b, res_e_w):
    row = edge_index[0].astype(jnp.int32)
    col = edge_index[1].astype(jnp.int32)

    w_node = jnp.concatenate([A_w, D_w, E_w, B_w], axis=0)
    b_node = jnp.concatenate([A_b, D_b, E_b, B_b], axis=0)[None]
    w_edge = jnp.concatenate([C_w, res_e_w], axis=0)
    b_edge = jnp.concatenate([C_b, jnp.zeros_like(C_b)], axis=0)[None]

    ax, dx, exbx = _nodeproj(x_input_node, w_node, b_node)
    cer = _edgeproj(edge_input_attr, w_edge, b_edge)
    eout, aggr = _sc_edge(row, col, dx, exbx, cer)
    x_final = _final(x_input_node, ax, aggr)
    return (x_final, eout)
